# baseline (speedup 1.0000x reference)
import jax
import jax.numpy as jnp
from jax import lax
from jax.experimental import pallas as pl
from jax.experimental.pallas import tpu as pltpu
from jax.experimental.pallas import tpu_sc as plsc

B, H, S, D = 8, 4, 2048, 128
SW = 512
BH = B * H
CACHE_ROWS = BH * S
SLABS = 2
TS = SLABS * S


def _tc_copy_body(k0, v0, k1, v1, out):
    out[0] = k0[...]
    out[1] = v0[...]
    out[2] = k1[...]
    out[3] = v1[...]


def _sc_patch(out_ref, lat_ref, pos_ref):
    mesh = plsc.ScalarSubcoreMesh(axis_name="c", num_cores=2)

    @pl.core_map(
        mesh,
        scratch_shapes=[pltpu.SMEM((24,), jnp.int32), pltpu.SemaphoreType.DMA],
    )
    def _(pos_s, sem):
        core = lax.axis_index("c")

        pltpu.sync_copy(pos_ref, pos_s)
        rcps = []
        for j in range(16):
            w = core * 16 + j
            p0 = pos_s[w // H]
            p1 = lax.bitwise_and(p0, SW - 1)
            row_lo = w * S
            for c in range(4):
                rcps.append(pltpu.make_async_copy(
                    lat_ref.at[pl.ds(c * BH + w, 1)],
                    out_ref.at[pl.ds(
                        c * CACHE_ROWS + row_lo + (p0 if c < 2 else p1), 1)],
                    sem,
                ))
        for cp in rcps:
            cp.start()
        for cp in rcps:
            cp.wait()


def kernel(k_cache_0, v_cache_0, k_cache_1, v_cache_1,
           latest_k_0, latest_v_0, latest_k_1, latest_v_1, position_ids):
    caches = [cc.reshape(BH * S, D)
              for cc in (k_cache_0, v_cache_0, k_cache_1, v_cache_1)]
    lat = jnp.stack([latest_k_0, latest_v_0, latest_k_1, latest_v_1],
                    axis=0).reshape(4 * BH, D)
    pos = jnp.pad(position_ids.reshape(B).astype(jnp.int32), (0, 16))

    out0 = pl.pallas_call(
        _tc_copy_body,
        grid=(BH // SLABS,),
        in_specs=[pl.BlockSpec((TS, D), lambda t: (t, 0))] * 4,
        out_specs=pl.BlockSpec((4, TS, D), lambda t: (0, t, 0)),
        out_shape=jax.ShapeDtypeStruct((4, CACHE_ROWS, D), jnp.float32),
    )(*caches)
    out0 = out0.reshape(4 * CACHE_ROWS, D)

    def upd(refs):
        out_ref, lat_ref, pos_ref = refs
        _sc_patch(out_ref, lat_ref, pos_ref)

    out1, _, _ = pl.run_state(upd)((out0, lat, pos))
    return out1.reshape(4, B, H, S, D)


# speedup vs baseline: 1.0132x; 1.0132x over previous
import jax
import jax.numpy as jnp
from jax import lax
from jax.experimental import pallas as pl
from jax.experimental.pallas import tpu as pltpu
from jax.experimental.pallas import tpu_sc as plsc

B, H, S, D = 8, 4, 2048, 128
SW = 512
BH = B * H
CACHE_ROWS = BH * S
SLABS = 2
TS = SLABS * S


def _tc_copy_body(k0, v0, k1, v1, out):
    out[0] = k0[...]
    out[1] = v0[...]
    out[2] = k1[...]
    out[3] = v1[...]


def _sc_patch(out_ref, lat_ref, pos_ref):
    mesh = plsc.ScalarSubcoreMesh(axis_name="c", num_cores=1)

    @pl.core_map(
        mesh,
        scratch_shapes=[pltpu.SMEM((24,), jnp.int32), pltpu.SemaphoreType.DMA],
    )
    def _(pos_s, sem):
        core = lax.axis_index("c")
        del core

        pltpu.sync_copy(pos_ref, pos_s)
        rcps = []
        for w in range(32):
            p0 = pos_s[w // H]
            p1 = lax.bitwise_and(p0, SW - 1)
            row_lo = w * S
            for c in range(4):
                rcps.append(pltpu.make_async_copy(
                    lat_ref.at[pl.ds(c * BH + w, 1)],
                    out_ref.at[pl.ds(
                        c * CACHE_ROWS + row_lo + (p0 if c < 2 else p1), 1)],
                    sem,
                ))
        for cp in rcps:
            cp.start()
        for cp in rcps:
            cp.wait()


def kernel(k_cache_0, v_cache_0, k_cache_1, v_cache_1,
           latest_k_0, latest_v_0, latest_k_1, latest_v_1, position_ids):
    caches = [cc.reshape(BH * S, D)
              for cc in (k_cache_0, v_cache_0, k_cache_1, v_cache_1)]
    lat = jnp.stack([latest_k_0, latest_v_0, latest_k_1, latest_v_1],
                    axis=0).reshape(4 * BH, D)
    pos = jnp.pad(position_ids.reshape(B).astype(jnp.int32), (0, 16))

    out0 = pl.pallas_call(
        _tc_copy_body,
        grid=(BH // SLABS,),
        in_specs=[pl.BlockSpec((TS, D), lambda t: (t, 0))] * 4,
        out_specs=pl.BlockSpec((4, TS, D), lambda t: (0, t, 0)),
        out_shape=jax.ShapeDtypeStruct((4, CACHE_ROWS, D), jnp.float32),
    )(*caches)
    out0 = out0.reshape(4 * CACHE_ROWS, D)

    def upd(refs):
        out_ref, lat_ref, pos_ref = refs
        _sc_patch(out_ref, lat_ref, pos_ref)

    out1, _, _ = pl.run_state(upd)((out0, lat, pos))
    return out1.reshape(4, B, H, S, D)
